# Initial kernel scaffold; baseline (speedup 1.0000x reference)
import jax
import jax.numpy as jnp
from jax import lax
from jax.experimental import pallas as pl
from jax.experimental.pallas import tpu as pltpu
from jax.experimental.pallas import tpu_sc as plsc

NC = 2
NS = 16
BN = 2048
KR = 8
F32 = jnp.float32


def _dot(a, b):
    return jnp.dot(a, b, preferred_element_type=F32)


def _stage_enc(xp, eW1T, eb1, eW2T, eb2, WlTs, WrT, bl):
    npad = xp.shape[0]
    nb = npad // BN

    def body(x_ref, w1_ref, b1_ref, w2_ref, b2_ref, wl_ref, wr_ref, bl_ref,
             p_ref, r_ref):
        x = x_ref[...]
        t = jnp.maximum(_dot(x[:, :32], w1_ref[...]) + b1_ref[...], 0.0)
        t = jnp.maximum(_dot(t, w2_ref[...]) + b2_ref[...], 0.0)
        h = jnp.concatenate([t, x[:, 32:]], axis=1)
        p_ref[0] = _dot(h, wl_ref[0])
        p_ref[1] = _dot(h, wl_ref[1])
        r_ref[...] = _dot(h, wr_ref[...]) + bl_ref[...]

    return pl.pallas_call(
        body,
        grid=(nb,),
        in_specs=[
            pl.BlockSpec((BN, 44), lambda i: (i, 0)),
            pl.BlockSpec((32, 32), lambda i: (0, 0)),
            pl.BlockSpec((1, 32), lambda i: (0, 0)),
            pl.BlockSpec((32, 32), lambda i: (0, 0)),
            pl.BlockSpec((1, 32), lambda i: (0, 0)),
            pl.BlockSpec((2, 44, 16), lambda i: (0, 0, 0)),
            pl.BlockSpec((44, 32), lambda i: (0, 0)),
            pl.BlockSpec((1, 32), lambda i: (0, 0)),
        ],
        out_specs=[
            pl.BlockSpec((2, BN, 16), lambda i: (0, i, 0)),
            pl.BlockSpec((BN, 32), lambda i: (i, 0)),
        ],
        out_shape=[
            jax.ShapeDtypeStruct((2, npad, 16), F32),
            jax.ShapeDtypeStruct((npad, 32), F32),
        ],
    )(xp, eW1T, eb1, eW2T, eb2, WlTs, WrT, bl)


def _stage_mid(agg, cnt, r_prev, WlTs, WrT, bl):
    npad = agg.shape[1]
    nb = npad // BN
    G = WlTs.shape[0]
    RW = WrT.shape[1]

    def body(agg_ref, cnt_ref, r_ref, wl_ref, wr_ref, bl_ref, p_ref, ro_ref):
        inv = 1.0 / jnp.maximum(cnt_ref[0] + cnt_ref[1], 1.0)
        m = jnp.concatenate([agg_ref[0] * inv, agg_ref[1] * inv], axis=1)
        h = jnp.maximum(m + r_ref[...], 0.0)
        for g in range(G):
            p_ref[g] = _dot(h, wl_ref[g])
        ro_ref[...] = _dot(h, wr_ref[...]) + bl_ref[...]

    return pl.pallas_call(
        body,
        grid=(nb,),
        in_specs=[
            pl.BlockSpec((2, BN, 16), lambda i: (0, i, 0)),
            pl.BlockSpec((2, BN, 1), lambda i: (0, i, 0)),
            pl.BlockSpec((BN, 32), lambda i: (i, 0)),
            pl.BlockSpec((G, 32, 16), lambda i: (0, 0, 0)),
            pl.BlockSpec((32, RW), lambda i: (0, 0)),
            pl.BlockSpec((1, RW), lambda i: (0, 0)),
        ],
        out_specs=[
            pl.BlockSpec((G, BN, 16), lambda i: (0, i, 0)),
            pl.BlockSpec((BN, RW), lambda i: (i, 0)),
        ],
        out_shape=[
            jax.ShapeDtypeStruct((G, npad, 16), F32),
            jax.ShapeDtypeStruct((npad, RW), F32),
        ],
    )(agg, cnt, r_prev, WlTs, WrT, bl)


def _stage_final(agg, cnt, r_prev):
    npad = agg.shape[1]
    nb = npad // BN

    def body(agg_ref, cnt_ref, r_ref, o_ref):
        inv = 1.0 / jnp.maximum(cnt_ref[0] + cnt_ref[1], 1.0)
        o_ref[...] = (agg_ref[0] + agg_ref[1]) * inv + r_ref[...]

    return pl.pallas_call(
        body,
        grid=(nb,),
        in_specs=[
            pl.BlockSpec((2, BN, 16), lambda i: (0, i, 0)),
            pl.BlockSpec((2, BN, 1), lambda i: (0, i, 0)),
            pl.BlockSpec((BN, 16), lambda i: (i, 0)),
        ],
        out_specs=pl.BlockSpec((BN, 16), lambda i: (i, 0)),
        out_shape=jax.ShapeDtypeStruct((npad, 16), F32),
    )(agg, cnt, r_prev)


def _sc_aggregate(p_tab, src2, dst2, npad, feature_split, with_cnt):
    er = src2.shape[0]
    nrt = npad // NS
    zfull, zrem = nrt // 1024, nrt % 1024
    CH = KR * 128

    if feature_split:
        rows_per_tile = er // NS
    else:
        rows_per_tile = (er // NC) // NS
    nchunks = rows_per_tile // KR

    out_type = [jax.ShapeDtypeStruct((2, npad, 16), F32)]
    if with_cnt:
        out_type.append(jax.ShapeDtypeStruct((2, npad), F32))

    scratch = [
        pltpu.VMEM_SHARED((npad, 16), F32),
        pltpu.VMEM_SHARED((npad,), F32),
        pltpu.VMEM((KR, 128), jnp.int32),
        pltpu.VMEM((KR, 128), jnp.int32),
        pltpu.VMEM((KR, 128), jnp.int32),
        pltpu.VMEM((CH, 16), F32),
        pltpu.VMEM((128,), F32),
        pltpu.VMEM((1024,), F32),
        pltpu.SemaphoreType.DMA,
    ]

    mesh = plsc.VectorSubcoreMesh(core_axis_name="c", subcore_axis_name="s")

    def body(*refs):
        if with_cnt:
            p_hbm, src_hbm, dst_hbm, agg_out, cnt_out = refs[:5]
            scr = refs[5:]
        else:
            p_hbm, src_hbm, dst_hbm, agg_out = refs[:4]
            cnt_out = None
            scr = refs[4:]
        agg_sp, cnt_sp, sbuf, dbuf, gbuf, vals, ones, fbuf, gsem = scr

        c = lax.axis_index("c")
        s = lax.axis_index("s")

        z16 = jnp.zeros((16,), F32)
        o16 = jnp.ones((16,), F32)

        def zrow(i, _):
            vals[i] = z16
            return 0
        lax.fori_loop(0, CH, zrow, 0)

        def zflat(i, _):
            fbuf[pl.ds(i * 16, 16)] = z16
            return 0
        lax.fori_loop(0, 64, zflat, 0)
        for t in range(8):
            ones[pl.ds(t * 16, 16)] = o16

        zbase = s * nrt
        for k in range(zfull):
            pltpu.sync_copy(vals.at[pl.ds(0, 1024)],
                            agg_sp.at[pl.ds(zbase + k * 1024, 1024)])
        if zrem:
            pltpu.sync_copy(vals.at[pl.ds(0, zrem)],
                            agg_sp.at[pl.ds(zbase + zfull * 1024, zrem)])
        if with_cnt:
            for k in range(zfull):
                pltpu.sync_copy(fbuf,
                                cnt_sp.at[pl.ds(zbase + k * 1024, 1024)])
            if zrem:
                pltpu.sync_copy(fbuf.at[pl.ds(0, zrem)],
                                cnt_sp.at[pl.ds(zbase + zfull * 1024, zrem)])
        plsc.subcore_barrier()

        if feature_split:
            tile_row0 = s * rows_per_tile
            goff = c * npad
        else:
            tile_row0 = c * (er // NC) + s * rows_per_tile
            goff = None

        def chunk(j, _):
            row0 = tile_row0 + j * KR
            pltpu.sync_copy(src_hbm.at[pl.ds(row0, KR)], sbuf)
            pltpu.sync_copy(dst_hbm.at[pl.ds(row0, KR)], dbuf)
            if feature_split:
                for t in range(KR):
                    for u in range(8):
                        gbuf[t, pl.ds(u * 16, 16)] = (
                            sbuf[t, pl.ds(u * 16, 16)] + goff)
                gidx = gbuf
            else:
                gidx = sbuf
            descs = [
                pltpu.async_copy(p_hbm.at[gidx.at[t]],
                                 vals.at[pl.ds(t * 128, 128)], gsem)
                for t in range(KR)
            ]
            for d in descs:
                d.wait()
            for t in range(KR):
                pltpu.sync_copy(vals.at[pl.ds(t * 128, 128)],
                                agg_sp.at[dbuf.at[t]], add=True)
            if with_cnt:
                for t in range(KR):
                    @pl.when(((row0 + t + c) % 2) == 0)
                    def _():
                        pltpu.sync_copy(ones, cnt_sp.at[dbuf.at[t]],
                                        add=True)
            return 0
        lax.fori_loop(0, nchunks, chunk, 0)
        plsc.subcore_barrier()

        obase = s * nrt
        pltpu.sync_copy(agg_sp.at[pl.ds(obase, nrt)],
                        agg_out.at[c].at[pl.ds(obase, nrt)])
        if with_cnt:
            pltpu.sync_copy(cnt_sp.at[pl.ds(obase, nrt)],
                            cnt_out.at[c].at[pl.ds(obase, nrt)])

    fn = pl.kernel(body, out_type=out_type, mesh=mesh, scratch_types=scratch)
    return fn(p_tab, src2, dst2)


def kernel(x, edge_index, eW1, eb1, eW2, eb2, W1l, b1l, W1r,
           W2l, b2l, W2r, W3l, b3l, W3r):
    n = x.shape[0]
    e = edge_index.shape[1]
    npad = ((n + BN - 1) // BN) * BN
    echunk = NS * KR * 128 * NC
    epad = ((e + echunk - 1) // echunk) * echunk

    xp = jnp.pad(x, ((0, npad - n), (0, 0)))

    pad_cnt = epad - e
    padidx = (n + (jnp.arange(pad_cnt, dtype=jnp.int32) % (npad - n - 1))
              ).astype(jnp.int32)
    src2 = jnp.concatenate([edge_index[0], padidx]).reshape(epad // 128, 128)
    dst2 = jnp.concatenate([edge_index[1], padidx]).reshape(epad // 128, 128)

    def split_wl(Wl):
        WlT = Wl.T
        return WlT.reshape(WlT.shape[0], 2, 16).transpose(1, 0, 2)

    p1, r1 = _stage_enc(
        xp, eW1.T, eb1.reshape(1, 32), eW2.T, eb2.reshape(1, 32),
        split_wl(W1l), W1r.T, b1l.reshape(1, 32))

    agg1, cnt = _sc_aggregate(p1.reshape(2 * npad, 16), src2, dst2, npad,
                              feature_split=True, with_cnt=True)
    cnt3 = cnt.reshape(2, npad, 1)

    p2, r2 = _stage_mid(agg1, cnt3, r1, split_wl(W2l), W2r.T,
                        b2l.reshape(1, 32))

    agg2 = _sc_aggregate(p2.reshape(2 * npad, 16), src2, dst2, npad,
                         feature_split=True, with_cnt=False)[0]

    W3lT = jnp.pad(W3l.T, ((0, 0), (0, 4))).reshape(1, 32, 16)
    W3rT = jnp.pad(W3r.T, ((0, 0), (0, 4)))
    b3p = jnp.pad(b3l, (0, 4)).reshape(1, 16)
    p3, r3 = _stage_mid(agg2, cnt3, r2, W3lT, W3rT, b3p)

    agg3 = _sc_aggregate(p3.reshape(npad, 16), src2, dst2, npad,
                         feature_split=False, with_cnt=False)[0]

    out = _stage_final(agg3, cnt3, r3)
    return out[:n, :12]


# speedup vs baseline: 14.9537x; 14.9537x over previous
import jax
import jax.numpy as jnp
from jax import lax
from jax.experimental import pallas as pl
from jax.experimental.pallas import tpu as pltpu
from jax.experimental.pallas import tpu_sc as plsc

NC = 2
NS = 16
BN = 2048
KR = 8
F32 = jnp.float32


def _dot(a, b):
    return jnp.dot(a, b, preferred_element_type=F32)


def _stage_enc(xp, eW1T, eb1, eW2T, eb2, WlTs, WrT, bl):
    npad = xp.shape[0]
    nb = npad // BN

    def body(x_ref, w1_ref, b1_ref, w2_ref, b2_ref, wl_ref, wr_ref, bl_ref,
             p_ref, r_ref):
        x = x_ref[...]
        t = jnp.maximum(_dot(x[:, :32], w1_ref[...]) + b1_ref[...], 0.0)
        t = jnp.maximum(_dot(t, w2_ref[...]) + b2_ref[...], 0.0)
        h = jnp.concatenate([t, x[:, 32:]], axis=1)
        p_ref[0] = _dot(h, wl_ref[0])
        p_ref[1] = _dot(h, wl_ref[1])
        r_ref[...] = _dot(h, wr_ref[...]) + bl_ref[...]

    return pl.pallas_call(
        body,
        grid=(nb,),
        in_specs=[
            pl.BlockSpec((BN, 44), lambda i: (i, 0)),
            pl.BlockSpec((32, 32), lambda i: (0, 0)),
            pl.BlockSpec((1, 32), lambda i: (0, 0)),
            pl.BlockSpec((32, 32), lambda i: (0, 0)),
            pl.BlockSpec((1, 32), lambda i: (0, 0)),
            pl.BlockSpec((2, 44, 16), lambda i: (0, 0, 0)),
            pl.BlockSpec((44, 32), lambda i: (0, 0)),
            pl.BlockSpec((1, 32), lambda i: (0, 0)),
        ],
        out_specs=[
            pl.BlockSpec((2, BN, 16), lambda i: (0, i, 0)),
            pl.BlockSpec((BN, 32), lambda i: (i, 0)),
        ],
        out_shape=[
            jax.ShapeDtypeStruct((2, npad, 16), F32),
            jax.ShapeDtypeStruct((npad, 32), F32),
        ],
    )(xp, eW1T, eb1, eW2T, eb2, WlTs, WrT, bl)


def _stage_mid(agg, cnt, r_prev, WlTs, WrT, bl):
    npad = agg.shape[1]
    nb = npad // BN
    G = WlTs.shape[0]
    RW = WrT.shape[1]

    def body(agg_ref, cnt_ref, r_ref, wl_ref, wr_ref, bl_ref, p_ref, ro_ref):
        inv = 1.0 / jnp.maximum(cnt_ref[0] + cnt_ref[1], 1.0)
        m = jnp.concatenate([agg_ref[0] * inv, agg_ref[1] * inv], axis=1)
        h = jnp.maximum(m + r_ref[...], 0.0)
        for g in range(G):
            p_ref[g] = _dot(h, wl_ref[g])
        ro_ref[...] = _dot(h, wr_ref[...]) + bl_ref[...]

    return pl.pallas_call(
        body,
        grid=(nb,),
        in_specs=[
            pl.BlockSpec((2, BN, 16), lambda i: (0, i, 0)),
            pl.BlockSpec((2, BN, 1), lambda i: (0, i, 0)),
            pl.BlockSpec((BN, 32), lambda i: (i, 0)),
            pl.BlockSpec((G, 32, 16), lambda i: (0, 0, 0)),
            pl.BlockSpec((32, RW), lambda i: (0, 0)),
            pl.BlockSpec((1, RW), lambda i: (0, 0)),
        ],
        out_specs=[
            pl.BlockSpec((G, BN, 16), lambda i: (0, i, 0)),
            pl.BlockSpec((BN, RW), lambda i: (i, 0)),
        ],
        out_shape=[
            jax.ShapeDtypeStruct((G, npad, 16), F32),
            jax.ShapeDtypeStruct((npad, RW), F32),
        ],
    )(agg, cnt, r_prev, WlTs, WrT, bl)


def _stage_final(agg, cnt, r_prev):
    npad = agg.shape[1]
    nb = npad // BN

    def body(agg_ref, cnt_ref, r_ref, o_ref):
        inv = 1.0 / jnp.maximum(cnt_ref[0] + cnt_ref[1], 1.0)
        o_ref[...] = (agg_ref[0] + agg_ref[1]) * inv + r_ref[...]

    return pl.pallas_call(
        body,
        grid=(nb,),
        in_specs=[
            pl.BlockSpec((2, BN, 16), lambda i: (0, i, 0)),
            pl.BlockSpec((2, BN, 1), lambda i: (0, i, 0)),
            pl.BlockSpec((BN, 16), lambda i: (i, 0)),
        ],
        out_specs=pl.BlockSpec((BN, 16), lambda i: (i, 0)),
        out_shape=jax.ShapeDtypeStruct((npad, 16), F32),
    )(agg, cnt, r_prev)


def _sc_aggregate(p_tab, src3, dst2, npad, feature_split, with_cnt):
    er = dst2.shape[0]
    nrt = npad // NS
    kr = 4 if with_cnt else 5
    ch = kr * 128
    zfull, zrem = nrt // ch, nrt % ch

    if feature_split:
        rows_per_tile = er // NS
    else:
        rows_per_tile = (er // NC) // NS
    nchunks = rows_per_tile // kr
    assert nchunks % 2 == 0 and rows_per_tile % kr == 0

    out_type = [jax.ShapeDtypeStruct((2, npad, 16), F32)]
    if with_cnt:
        out_type.append(jax.ShapeDtypeStruct((2, npad), F32))

    scratch = [
        pltpu.VMEM_SHARED((npad, 16), F32),
        pltpu.VMEM_SHARED((npad if with_cnt else 8,), F32),
        pltpu.VMEM((2, kr, 128), jnp.int32),
        pltpu.VMEM((2, kr, 128), jnp.int32),
        pltpu.VMEM((2, ch, 16), F32),
        pltpu.VMEM((128,), F32),
        pltpu.VMEM((1024,), F32),
        pltpu.SemaphoreType.DMA((2,)),
        pltpu.SemaphoreType.DMA((2,)),
        pltpu.SemaphoreType.DMA((2,)),
        pltpu.SemaphoreType.DMA((2,)),
    ]

    mesh = plsc.VectorSubcoreMesh(core_axis_name="c", subcore_axis_name="s")

    def body(*refs):
        if with_cnt:
            p_hbm, src_hbm, dst_hbm, agg_out, cnt_out = refs[:5]
            scr = refs[5:]
        else:
            p_hbm, src_hbm, dst_hbm, agg_out = refs[:4]
            cnt_out = None
            scr = refs[4:]
        (agg_sp, cnt_sp, sbuf, dbuf, vals, ones, fbuf,
         isems, gsems, ssems, csems) = scr

        c = lax.axis_index("c")
        s = lax.axis_index("s")

        z16 = jnp.zeros((16,), F32)
        o16 = jnp.ones((16,), F32)

        def zrow(i, _):
            vals[0, i] = z16
            return 0
        lax.fori_loop(0, ch, zrow, 0)

        def zflat(i, _):
            fbuf[pl.ds(i * 16, 16)] = z16
            return 0
        lax.fori_loop(0, 64, zflat, 0)
        for t in range(8):
            ones[pl.ds(t * 16, 16)] = o16

        zbase = s * nrt
        for k in range(zfull):
            pltpu.sync_copy(vals.at[0].at[pl.ds(0, ch)],
                            agg_sp.at[pl.ds(zbase + k * ch, ch)])
        if zrem:
            pltpu.sync_copy(vals.at[0].at[pl.ds(0, zrem)],
                            agg_sp.at[pl.ds(zbase + zfull * ch, zrem)])
        if with_cnt:
            for k in range(nrt // 1024):
                pltpu.sync_copy(fbuf,
                                cnt_sp.at[pl.ds(zbase + k * 1024, 1024)])
            crem = nrt % 1024
            if crem:
                pltpu.sync_copy(fbuf.at[pl.ds(0, crem)],
                                cnt_sp.at[pl.ds(zbase + (nrt // 1024) * 1024,
                                                crem)])
        plsc.subcore_barrier()

        if feature_split:
            tile_row0 = s * rows_per_tile
        else:
            tile_row0 = c * (er // NC) + s * rows_per_tile

        def start_idx(j, b):
            row0 = tile_row0 + j * kr
            pltpu.async_copy(src_hbm.at[c].at[pl.ds(row0, kr)],
                             sbuf.at[b], isems.at[b])
            pltpu.async_copy(dst_hbm.at[pl.ds(row0, kr)],
                             dbuf.at[b], isems.at[b])

        def wait_idx(b):
            for _ in range(2):
                pltpu.make_async_copy(dst_hbm.at[pl.ds(0, kr)],
                                      sbuf.at[b], isems.at[b]).wait()

        def drain_scatter(b):
            for t in range(kr):
                pltpu.make_async_copy(
                    vals.at[b].at[pl.ds(t * 128, 128)],
                    agg_sp.at[dbuf.at[b].at[t]], ssems.at[b]).wait()

        def drain_cnt(b, row0):
            for t in range(kr):
                @pl.when(((row0 + t + c) % 2) == 0)
                def _():
                    pltpu.make_async_copy(
                        ones, cnt_sp.at[dbuf.at[b].at[t]],
                        csems.at[b]).wait()

        start_idx(0, 0)

        def pipe(jj, _):
            for b in (0, 1):
                j = 2 * jj + b
                row0 = tile_row0 + j * kr
                wait_idx(b)
                gds = [
                    pltpu.async_copy(p_hbm.at[sbuf.at[b].at[t]],
                                     vals.at[b].at[pl.ds(t * 128, 128)],
                                     gsems.at[b])
                    for t in range(kr)
                ]

                @pl.when(j >= 1)
                def _():
                    drain_scatter(1 - b)
                    if with_cnt:
                        drain_cnt(1 - b, row0)

                @pl.when(j + 1 < nchunks)
                def _():
                    start_idx(j + 1, 1 - b)
                for d in gds:
                    d.wait()
                for t in range(kr):
                    pltpu.async_copy(vals.at[b].at[pl.ds(t * 128, 128)],
                                     agg_sp.at[dbuf.at[b].at[t]],
                                     ssems.at[b], add=True)
                if with_cnt:
                    for t in range(kr):
                        @pl.when(((row0 + t + c) % 2) == 0)
                        def _():
                            pltpu.async_copy(ones,
                                             cnt_sp.at[dbuf.at[b].at[t]],
                                             csems.at[b], add=True)
            return 0
        lax.fori_loop(0, nchunks // 2, pipe, 0)
        drain_scatter(1)
        if with_cnt:
            drain_cnt(1, tile_row0 + (nchunks - 1) * kr)
        plsc.subcore_barrier()

        obase = s * nrt
        pltpu.sync_copy(agg_sp.at[pl.ds(obase, nrt)],
                        agg_out.at[c].at[pl.ds(obase, nrt)])
        if with_cnt:
            pltpu.sync_copy(cnt_sp.at[pl.ds(obase, nrt)],
                            cnt_out.at[c].at[pl.ds(obase, nrt)])

    fn = pl.kernel(body, out_type=out_type, mesh=mesh, scratch_types=scratch,
                   compiler_params=pltpu.CompilerParams(
                       use_tc_tiling_on_sc=False))
    return fn(p_tab, src3, dst2)


def kernel(x, edge_index, eW1, eb1, eW2, eb2, W1l, b1l, W1r,
           W2l, b2l, W2r, W3l, b3l, W3r):
    n = x.shape[0]
    e = edge_index.shape[1]
    npad = ((n + BN - 1) // BN) * BN
    echunk = 65536
    epad = ((e + echunk - 1) // echunk) * echunk

    xp = jnp.pad(x, ((0, npad - n), (0, 0)))

    pad_cnt = epad - e
    padidx = (n + (jnp.arange(pad_cnt, dtype=jnp.int32) % (npad - n - 1))
              ).astype(jnp.int32)
    src = jnp.concatenate([edge_index[0], padidx]).reshape(epad // 128, 128)
    src3 = jnp.stack([src, src + npad])
    src3e = jnp.stack([src, src])
    dst2 = jnp.concatenate([edge_index[1], padidx]).reshape(epad // 128, 128)

    def split_wl(Wl):
        WlT = Wl.T
        return WlT.reshape(WlT.shape[0], 2, 16).transpose(1, 0, 2)

    p1, r1 = _stage_enc(
        xp, eW1.T, eb1.reshape(1, 32), eW2.T, eb2.reshape(1, 32),
        split_wl(W1l), W1r.T, b1l.reshape(1, 32))

    agg1, cnt = _sc_aggregate(p1.reshape(2 * npad, 16), src3, dst2, npad,
                              feature_split=True, with_cnt=True)
    cnt3 = cnt.reshape(2, npad, 1)

    p2, r2 = _stage_mid(agg1, cnt3, r1, split_wl(W2l), W2r.T,
                        b2l.reshape(1, 32))

    agg2 = _sc_aggregate(p2.reshape(2 * npad, 16), src3, dst2, npad,
                         feature_split=True, with_cnt=False)[0]

    W3lT = jnp.pad(W3l.T, ((0, 0), (0, 4))).reshape(1, 32, 16)
    W3rT = jnp.pad(W3r.T, ((0, 0), (0, 4)))
    b3p = jnp.pad(b3l, (0, 4)).reshape(1, 16)
    p3, r3 = _stage_mid(agg2, cnt3, r2, W3lT, W3rT, b3p)

    agg3 = _sc_aggregate(p3.reshape(npad, 16), src3e, dst2, npad,
                         feature_split=False, with_cnt=False)[0]

    out = _stage_final(agg3, cnt3, r3)
    return out[:n, :12]


# speedup vs baseline: 15.1901x; 1.0158x over previous
import jax
import jax.numpy as jnp
from jax import lax
from jax.experimental import pallas as pl
from jax.experimental.pallas import tpu as pltpu
from jax.experimental.pallas import tpu_sc as plsc

NC = 2
NS = 16
BN = 2048
KR = 4
F32 = jnp.float32


def _dot(a, b):
    return jnp.dot(a, b, preferred_element_type=F32)


def _stage_enc(x, npad, eW1T, eb1, eW2T, eb2, WlTs, WrT, bl):
    nb = npad // BN

    def body(x_ref, w1_ref, b1_ref, w2_ref, b2_ref, wl_ref, wr_ref, bl_ref,
             p_ref, r_ref):
        x_ = x_ref[...]
        t = jnp.maximum(_dot(x_[:, :32], w1_ref[...]) + b1_ref[...], 0.0)
        t = jnp.maximum(_dot(t, w2_ref[...]) + b2_ref[...], 0.0)
        h = jnp.concatenate([t, x_[:, 32:]], axis=1)
        p_ref[0] = _dot(h, wl_ref[0])
        p_ref[1] = _dot(h, wl_ref[1])
        r_ref[...] = _dot(h, wr_ref[...]) + bl_ref[...]

    return pl.pallas_call(
        body,
        grid=(nb,),
        in_specs=[
            pl.BlockSpec((BN, 44), lambda i: (i, 0)),
            pl.BlockSpec((32, 32), lambda i: (0, 0)),
            pl.BlockSpec((1, 32), lambda i: (0, 0)),
            pl.BlockSpec((32, 32), lambda i: (0, 0)),
            pl.BlockSpec((1, 32), lambda i: (0, 0)),
            pl.BlockSpec((2, 44, 16), lambda i: (0, 0, 0)),
            pl.BlockSpec((44, 32), lambda i: (0, 0)),
            pl.BlockSpec((1, 32), lambda i: (0, 0)),
        ],
        out_specs=[
            pl.BlockSpec((2, BN, 16), lambda i: (0, i, 0)),
            pl.BlockSpec((BN, 32), lambda i: (i, 0)),
        ],
        out_shape=[
            jax.ShapeDtypeStruct((2, npad, 16), F32),
            jax.ShapeDtypeStruct((npad, 32), F32),
        ],
    )(x, eW1T, eb1, eW2T, eb2, WlTs, WrT, bl)


def _stage_mid(agg, cnt_rep, r_prev, WlTs, WrT, bl):
    npad = agg.shape[1]
    nb = npad // BN
    G = WlTs.shape[0]
    RW = WrT.shape[1]

    def body(agg_ref, cnt_ref, r_ref, wl_ref, wr_ref, bl_ref, p_ref, ro_ref):
        inv = 1.0 / jnp.maximum(cnt_ref[0] + cnt_ref[1], 1.0)
        m = jnp.concatenate([agg_ref[0] * inv, agg_ref[1] * inv], axis=1)
        h = jnp.maximum(m + r_ref[...], 0.0)
        for g in range(G):
            p_ref[g] = _dot(h, wl_ref[g])
        ro_ref[...] = _dot(h, wr_ref[...]) + bl_ref[...]

    return pl.pallas_call(
        body,
        grid=(nb,),
        in_specs=[
            pl.BlockSpec((2, BN, 16), lambda i: (0, i, 0)),
            pl.BlockSpec((2, BN, 16), lambda i: (0, i, 0)),
            pl.BlockSpec((BN, 32), lambda i: (i, 0)),
            pl.BlockSpec((G, 32, 16), lambda i: (0, 0, 0)),
            pl.BlockSpec((32, RW), lambda i: (0, 0)),
            pl.BlockSpec((1, RW), lambda i: (0, 0)),
        ],
        out_specs=[
            pl.BlockSpec((G, BN, 16), lambda i: (0, i, 0)),
            pl.BlockSpec((BN, RW), lambda i: (i, 0)),
        ],
        out_shape=[
            jax.ShapeDtypeStruct((G, npad, 16), F32),
            jax.ShapeDtypeStruct((npad, RW), F32),
        ],
    )(agg, cnt_rep, r_prev, WlTs, WrT, bl)


def _stage_final(agg, cnt_rep, r_prev, n):
    npad = agg.shape[1]
    nb = npad // BN

    def body(agg_ref, cnt_ref, r_ref, o_ref):
        inv = 1.0 / jnp.maximum(cnt_ref[0] + cnt_ref[1], 1.0)
        o_ref[...] = ((agg_ref[0] + agg_ref[1]) * inv + r_ref[...])[:, :12]

    return pl.pallas_call(
        body,
        grid=(nb,),
        in_specs=[
            pl.BlockSpec((2, BN, 16), lambda i: (0, i, 0)),
            pl.BlockSpec((2, BN, 16), lambda i: (0, i, 0)),
            pl.BlockSpec((BN, 16), lambda i: (i, 0)),
        ],
        out_specs=pl.BlockSpec((BN, 12), lambda i: (i, 0)),
        out_shape=jax.ShapeDtypeStruct((n, 12), F32),
    )(agg, cnt_rep, r_prev)


def _sc_aggregate(p_tab, src2, dst2, npad, feature_split, with_cnt):
    er = dst2.shape[0]
    nrt = npad // NS
    kr = KR
    ch = kr * 128
    zfull, zrem = nrt // ch, nrt % ch

    if feature_split:
        rows_per_tile = er // NS
    else:
        rows_per_tile = (er // NC) // NS
    nchunks = rows_per_tile // kr
    assert nchunks % 2 == 0 and rows_per_tile % kr == 0
    assert nrt % 128 == 0

    out_type = [jax.ShapeDtypeStruct((2, npad, 16), F32)]
    if with_cnt:
        out_type.append(jax.ShapeDtypeStruct((2, npad, 16), F32))

    scratch = [
        pltpu.VMEM_SHARED((npad, 16), F32),
        pltpu.VMEM_SHARED((npad if with_cnt else 8,), F32),
        pltpu.VMEM((2, kr, 128), jnp.int32),
        pltpu.VMEM((2, kr, 128), jnp.int32),
        pltpu.VMEM((kr, 128), jnp.int32),
        pltpu.VMEM((2, ch, 16), F32),
        pltpu.VMEM((128,), F32),
        pltpu.VMEM((128,), F32),
        pltpu.VMEM((128, 16), F32),
        pltpu.SemaphoreType.DMA((2,)),
        pltpu.SemaphoreType.DMA((2,)),
        pltpu.SemaphoreType.DMA((2,)),
        pltpu.SemaphoreType.DMA((2,)),
    ]

    mesh = plsc.VectorSubcoreMesh(core_axis_name="c", subcore_axis_name="s")

    def body(*refs):
        if with_cnt:
            p_hbm, src_hbm, dst_hbm, agg_out, cnt_out = refs[:5]
            scr = refs[5:]
        else:
            p_hbm, src_hbm, dst_hbm, agg_out = refs[:4]
            cnt_out = None
            scr = refs[4:]
        (agg_sp, cnt_sp, sbuf, dbuf, gbuf, vals, ones, fbuf, cpack,
         isems, gsems, ssems, csems) = scr

        c = lax.axis_index("c")
        s = lax.axis_index("s")

        z16 = jnp.zeros((16,), F32)
        o16 = jnp.ones((16,), F32)

        def zrow(i, _):
            vals[0, i] = z16
            return 0
        lax.fori_loop(0, ch, zrow, 0)
        for t in range(8):
            fbuf[pl.ds(t * 16, 16)] = z16
            ones[pl.ds(t * 16, 16)] = o16

        zbase = s * nrt
        for k in range(zfull):
            pltpu.sync_copy(vals.at[0].at[pl.ds(0, ch)],
                            agg_sp.at[pl.ds(zbase + k * ch, ch)])
        if zrem:
            pltpu.sync_copy(vals.at[0].at[pl.ds(0, zrem)],
                            agg_sp.at[pl.ds(zbase + zfull * ch, zrem)])
        if with_cnt:
            for k in range(nrt // 128):
                pltpu.sync_copy(fbuf,
                                cnt_sp.at[pl.ds(zbase + k * 128, 128)])
        plsc.subcore_barrier()

        if feature_split:
            tile_row0 = s * rows_per_tile
        else:
            tile_row0 = c * (er // NC) + s * rows_per_tile
        goff = c * npad

        def start_idx(j, b):
            row0 = tile_row0 + j * kr
            pltpu.async_copy(src_hbm.at[pl.ds(row0, kr)],
                             sbuf.at[b], isems.at[b])
            pltpu.async_copy(dst_hbm.at[pl.ds(row0, kr)],
                             dbuf.at[b], isems.at[b])

        def wait_idx(b):
            for _ in range(2):
                pltpu.make_async_copy(dst_hbm.at[pl.ds(0, kr)],
                                      sbuf.at[b], isems.at[b]).wait()

        def drain_scatter(b):
            for t in range(kr):
                pltpu.make_async_copy(
                    vals.at[b].at[pl.ds(t * 128, 128)],
                    agg_sp.at[dbuf.at[b].at[t]], ssems.at[b]).wait()

        def drain_cnt(b, row0):
            for t in range(kr):
                @pl.when(((row0 + t + c) % 2) == 0)
                def _():
                    pltpu.make_async_copy(
                        ones, cnt_sp.at[dbuf.at[b].at[t]],
                        csems.at[b]).wait()

        start_idx(0, 0)

        def pipe(jj, _):
            for b in (0, 1):
                j = 2 * jj + b
                row0 = tile_row0 + j * kr
                wait_idx(b)
                if feature_split:
                    for t in range(kr):
                        for u in range(8):
                            gbuf[t, pl.ds(u * 16, 16)] = (
                                sbuf[b, t, pl.ds(u * 16, 16)] + goff)
                    gidx = gbuf
                else:
                    gidx = sbuf.at[b]
                gds = [
                    pltpu.async_copy(p_hbm.at[gidx.at[t]],
                                     vals.at[b].at[pl.ds(t * 128, 128)],
                                     gsems.at[b])
                    for t in range(kr)
                ]

                @pl.when(j >= 1)
                def _():
                    drain_scatter(1 - b)
                    if with_cnt:
                        drain_cnt(1 - b, row0)

                @pl.when(j + 1 < nchunks)
                def _():
                    start_idx(j + 1, 1 - b)
                for d in gds:
                    d.wait()
                for t in range(kr):
                    pltpu.async_copy(vals.at[b].at[pl.ds(t * 128, 128)],
                                     agg_sp.at[dbuf.at[b].at[t]],
                                     ssems.at[b], add=True)
                if with_cnt:
                    for t in range(kr):
                        @pl.when(((row0 + t + c) % 2) == 0)
                        def _():
                            pltpu.async_copy(ones,
                                             cnt_sp.at[dbuf.at[b].at[t]],
                                             csems.at[b], add=True)
            return 0
        lax.fori_loop(0, nchunks // 2, pipe, 0)
        drain_scatter(1)
        if with_cnt:
            drain_cnt(1, tile_row0 + (nchunks - 1) * kr)
        plsc.subcore_barrier()

        obase = s * nrt
        pltpu.sync_copy(agg_sp.at[pl.ds(obase, nrt)],
                        agg_out.at[c].at[pl.ds(obase, nrt)])
        if with_cnt:
            def crep(o, _):
                pltpu.sync_copy(cnt_sp.at[pl.ds(zbase + o * 128, 128)],
                                fbuf)
                for g in range(8):
                    v = fbuf[pl.ds(g * 16, 16)]
                    for l in range(16):
                        cpack[g * 16 + l] = jnp.broadcast_to(v[l], (16,))
                pltpu.sync_copy(
                    cpack,
                    cnt_out.at[c].at[pl.ds(zbase + o * 128, 128)])
                return 0
            lax.fori_loop(0, nrt // 128, crep, 0)

    fn = pl.kernel(body, out_type=out_type, mesh=mesh, scratch_types=scratch,
                   compiler_params=pltpu.CompilerParams(
                       use_tc_tiling_on_sc=False))
    return fn(p_tab, src2, dst2)


def kernel(x, edge_index, eW1, eb1, eW2, eb2, W1l, b1l, W1r,
           W2l, b2l, W2r, W3l, b3l, W3r):
    n = x.shape[0]
    e = edge_index.shape[1]
    npad = ((n + BN - 1) // BN) * BN
    echunk = NS * KR * 128 * NC
    epad = ((e + echunk - 1) // echunk) * echunk

    pad_cnt = epad - e
    padidx = (n + (jnp.arange(pad_cnt, dtype=jnp.int32) % (npad - n - 1))
              ).astype(jnp.int32)
    src2 = jnp.concatenate([edge_index[0], padidx]).reshape(epad // 128, 128)
    dst2 = jnp.concatenate([edge_index[1], padidx]).reshape(epad // 128, 128)

    def split_wl(Wl):
        WlT = Wl.T
        return WlT.reshape(WlT.shape[0], 2, 16).transpose(1, 0, 2)

    p1, r1 = _stage_enc(
        x, npad, eW1.T, eb1.reshape(1, 32), eW2.T, eb2.reshape(1, 32),
        split_wl(W1l), W1r.T, b1l.reshape(1, 32))

    agg1, cnt_rep = _sc_aggregate(p1.reshape(2 * npad, 16), src2, dst2, npad,
                                  feature_split=True, with_cnt=True)

    p2, r2 = _stage_mid(agg1, cnt_rep, r1, split_wl(W2l), W2r.T,
                        b2l.reshape(1, 32))

    agg2 = _sc_aggregate(p2.reshape(2 * npad, 16), src2, dst2, npad,
                         feature_split=True, with_cnt=False)[0]

    W3lT = jnp.pad(W3l.T, ((0, 0), (0, 4))).reshape(1, 32, 16)
    W3rT = jnp.pad(W3r.T, ((0, 0), (0, 4)))
    b3p = jnp.pad(b3l, (0, 4)).reshape(1, 16)
    p3, r3 = _stage_mid(agg2, cnt_rep, r2, W3lT, W3rT, b3p)

    agg3 = _sc_aggregate(p3.reshape(npad, 16), src2, dst2, npad,
                         feature_split=False, with_cnt=False)[0]

    return _stage_final(agg3, cnt_rep, r3, n)
